# baseline (speedup 1.0000x reference)
import functools
import math

import jax
import jax.numpy as jnp
from jax import lax
from jax.experimental import pallas as pl
from jax.experimental.pallas import tpu as pltpu
from jax.experimental.pallas import tpu_sc as plsc

VOCAB = 1000000
DIM = 64
BATCH = 4096
SEQ = 200
SCALE = math.sqrt(DIM)

NC = 2
NS = 16
NW = NC * NS
LANES = 16
KSUB = DIM // LANES
PDIM = 128

TBLK = 8192
NTBLK = (VOCAB + TBLK - 1) // TBLK

BBLK = 512
NE = BATCH // BBLK
N_UNITS = SEQ * NE
UNITS_PW = N_UNITS // NW
GSUB = 64
NG = BBLK // GSUB
NA = 4
BPITCH = BBLK + 1
SB = SEQ // 8
BB = BATCH // 128


def _prep_table_block(tt_ref, out_ref):
    out_ref[:, 0:DIM] = (jnp.transpose(tt_ref[...]) * SCALE).astype(jnp.bfloat16)


_prep_table = pl.pallas_call(
    _prep_table_block,
    grid=(NTBLK,),
    in_specs=[pl.BlockSpec((DIM, TBLK), lambda i: (0, i))],
    out_specs=pl.BlockSpec((TBLK, PDIM), lambda i: (i, 0)),
    out_shape=jax.ShapeDtypeStruct((VOCAB, PDIM), jnp.bfloat16),
)


def _iota16():
    return lax.iota(jnp.int32, LANES)


@functools.partial(
    pl.kernel,
    out_type=jax.ShapeDtypeStruct((SEQ, 8, BB, 8, 128), jnp.float32),
    mesh=plsc.VectorSubcoreMesh(core_axis_name="c", subcore_axis_name="s"),
    scratch_types=[
        [pltpu.VMEM((BBLK,), jnp.int32) for _ in range(2)],
        [pltpu.VMEM((GSUB, PDIM), jnp.bfloat16) for _ in range(NA)],
        [pltpu.VMEM((8, 8, BPITCH), jnp.float32) for _ in range(2)],
        [pltpu.SemaphoreType.DMA for _ in range(2)],
        [pltpu.SemaphoreType.DMA for _ in range(NA)],
        [pltpu.SemaphoreType.DMA for _ in range(2)],
    ],
    compiler_params=pltpu.CompilerParams(use_tc_tiling_on_sc=False,
                                         needs_layout_passes=False),
)
def _gather_t(xq_hbm, table_hbm, out_hbm, idxbufs, abufs, bbufs,
              isems, gsems, osems):
    wid = lax.axis_index("s") * NC + lax.axis_index("c")

    def unit_su(u):
        return u // NE, u % NE

    def start_idx(u, slot):
        s, e = unit_su(u)
        sb, si = s // 8, s % 8
        for q in range(4):
            pltpu.async_copy(
                xq_hbm.at[sb, e * 4 + q, si, :],
                idxbufs[slot].at[pl.ds(q * 128, 128)],
                isems[slot],
            )

    def wait_idx(slot):
        for _ in range(4):
            pltpu.make_async_copy(
                xq_hbm.at[0, 0, 0, :],
                idxbufs[slot].at[pl.ds(0, 128)],
                isems[slot],
            ).wait()

    def start_gather(islot, sub, aslot):
        pltpu.async_copy(
            table_hbm.at[idxbufs[islot].at[pl.ds(sub * GSUB, GSUB)]],
            abufs[aslot],
            gsems[aslot],
        )

    def wait_gather(aslot):
        pltpu.make_async_copy(
            table_hbm.at[idxbufs[0].at[pl.ds(0, GSUB)]],
            abufs[aslot],
            gsems[aslot],
        ).wait()

    def start_out(u, bslot):
        s, e = unit_su(u)
        for q in range(4):
            pltpu.async_copy(
                bbufs[bslot].at[:, :, pl.ds(q * 128, 128)],
                out_hbm.at[s, :, e * 4 + q, :, :],
                osems[bslot],
            )

    def wait_out(bslot):
        for _ in range(4):
            pltpu.make_async_copy(
                bbufs[bslot].at[:, :, pl.ds(0, 128)],
                out_hbm.at[0, :, 0, :, :],
                osems[bslot],
            ).wait()

    iota = _iota16()
    d_even = [32 * g + 2 * iota for g in range(2)]
    d_odd = [32 * g + 1 + 2 * iota for g in range(2)]
    dbs = [d // 8 for d in d_even + d_odd]
    dis = [d % 8 for d in d_even + d_odd]

    def transpose_block(aslot, bslot, sub):
        ab = abufs[aslot]
        bb = bbufs[bslot]
        base = jnp.full((LANES,), sub * GSUB, jnp.int32)

        @plsc.parallel_loop(0, GSUB, step=4, unroll=2)
        def _body(r):
            col = base + r
            for rr in range(4):
                for g in range(2):
                    v = ab[r + rr, pl.ds(g * 32, 32)]
                    lo, hi = plsc.unpack(v, format=plsc.PackFormat.INTERLEAVED)
                    plsc.store_scatter(bb, [dbs[g], dis[g], col + rr], lo)
                    plsc.store_scatter(bb, [dbs[2 + g], dis[2 + g], col + rr], hi)

    u0 = wid * UNITS_PW
    start_idx(u0, 0)
    wait_idx(0)
    for j in range(3):
        start_gather(0, j, j)

    def outer(k2, carry):
        for p in range(2):
            k = k2 * 2 + p
            u = u0 + k

            @pl.when(k + 1 < UNITS_PW)
            def _():
                start_idx(u + 1, 1 - p)

            @pl.when(k >= 2)
            def _():
                wait_out(p)

            for sub in range(NG):
                aslot = sub % NA
                pre = sub + 3
                if pre < NG:
                    start_gather(p, pre, pre % NA)
                else:
                    @pl.when(k + 1 < UNITS_PW)
                    def _(pre=pre):
                        if pre == NG:
                            wait_idx(1 - p)
                        start_gather(1 - p, pre - NG, pre % NA)

                wait_gather(aslot)
                transpose_block(aslot, p, sub)

            start_out(u, p)
        return carry

    lax.fori_loop(0, UNITS_PW // 2, outer, 0)

    for j in (UNITS_PW - 2, UNITS_PW - 1):
        wait_out(j % 2)


def kernel(x, table):
    xq = (
        x.astype(jnp.int32)
        .reshape(BB, 128, SB, 8)
        .transpose(2, 0, 3, 1)
    )
    t128 = _prep_table(table.T)
    out5 = _gather_t(xq, t128)
    return out5.transpose(2, 4, 0, 1, 3).reshape(BATCH, SEQ, DIM)


# speedup vs baseline: 2.4015x; 2.4015x over previous
import functools
import math

import jax
import jax.numpy as jnp
from jax import lax
from jax.experimental import pallas as pl
from jax.experimental.pallas import tpu as pltpu
from jax.experimental.pallas import tpu_sc as plsc

VOCAB = 1000000
DIM = 64
BATCH = 4096
SEQ = 200
SCALE = math.sqrt(DIM)

NC = 2
NS = 16
NW = NC * NS
LANES = 16
KSUB = DIM // LANES
PDIM = 128

TBLK = 16384
NTBLK = (VOCAB + TBLK - 1) // TBLK

BBLK = 512
NE = BATCH // BBLK
N_UNITS = SEQ * NE
UNITS_PW = N_UNITS // NW
GSUB = 64
NG = BBLK // GSUB
NA = 4
BPITCH = BBLK + 1
SB = SEQ // 8
BB = BATCH // 128


def _prep_table_block(tt_ref, out_ref):
    out_ref[:, 0:DIM] = jnp.transpose(tt_ref[...]) * SCALE


_prep_table = pl.pallas_call(
    _prep_table_block,
    grid=(NTBLK,),
    in_specs=[pl.BlockSpec((DIM, TBLK), lambda i: (0, i))],
    out_specs=pl.BlockSpec((TBLK, PDIM), lambda i: (i, 0)),
    out_shape=jax.ShapeDtypeStruct((VOCAB, PDIM), jnp.float32),
)


def _iota16():
    return lax.iota(jnp.int32, LANES)


@functools.partial(
    pl.kernel,
    out_type=jax.ShapeDtypeStruct((SEQ, 8, BB, 8, 128), jnp.float32),
    mesh=plsc.VectorSubcoreMesh(core_axis_name="c", subcore_axis_name="s"),
    scratch_types=[
        [pltpu.VMEM((BBLK,), jnp.int32) for _ in range(2)],
        [pltpu.VMEM((GSUB, PDIM), jnp.float32) for _ in range(NA)],
        [pltpu.VMEM((8, 8, BPITCH), jnp.float32) for _ in range(2)],
        [pltpu.SemaphoreType.DMA for _ in range(2)],
        [pltpu.SemaphoreType.DMA for _ in range(NA)],
        [pltpu.SemaphoreType.DMA for _ in range(2)],
    ],
    compiler_params=pltpu.CompilerParams(use_tc_tiling_on_sc=False,
                                         needs_layout_passes=False),
)
def _gather_t(xq_hbm, table_hbm, out_hbm, idxbufs, abufs, bbufs,
              isems, gsems, osems):
    wid = lax.axis_index("s") * NC + lax.axis_index("c")

    def unit_su(u):
        return u // NE, u % NE

    def start_idx(u, slot):
        s, e = unit_su(u)
        sb, si = s // 8, s % 8
        for q in range(4):
            pltpu.async_copy(
                xq_hbm.at[sb, e * 4 + q, si, :],
                idxbufs[slot].at[pl.ds(q * 128, 128)],
                isems[slot],
            )

    def wait_idx(slot):
        for _ in range(4):
            pltpu.make_async_copy(
                xq_hbm.at[0, 0, 0, :],
                idxbufs[slot].at[pl.ds(0, 128)],
                isems[slot],
            ).wait()

    def start_gather(islot, sub, aslot):
        pltpu.async_copy(
            table_hbm.at[idxbufs[islot].at[pl.ds(sub * GSUB, GSUB)]],
            abufs[aslot],
            gsems[aslot],
        )

    def wait_gather(aslot):
        pltpu.make_async_copy(
            table_hbm.at[idxbufs[0].at[pl.ds(0, GSUB)]],
            abufs[aslot],
            gsems[aslot],
        ).wait()

    def start_out(u, bslot):
        s, e = unit_su(u)
        for q in range(4):
            pltpu.async_copy(
                bbufs[bslot].at[:, :, pl.ds(q * 128, 128)],
                out_hbm.at[s, :, e * 4 + q, :, :],
                osems[bslot],
            )

    def wait_out(bslot):
        for _ in range(4):
            pltpu.make_async_copy(
                bbufs[bslot].at[:, :, pl.ds(0, 128)],
                out_hbm.at[0, :, 0, :, :],
                osems[bslot],
            ).wait()

    iota = _iota16()
    dbs = [iota // 8 + 2 * k for k in range(KSUB)]
    dis = [iota % 8 for _ in range(KSUB)]

    def transpose_block(aslot, bslot, sub):
        ab = abufs[aslot]
        bb = bbufs[bslot]
        base = jnp.full((LANES,), sub * GSUB, jnp.int32)

        @plsc.parallel_loop(0, GSUB, step=4, unroll=4)
        def _body(r):
            col = base + r
            for rr in range(4):
                for k in range(KSUB):
                    v = ab[r + rr, pl.ds(k * LANES, LANES)]
                    plsc.store_scatter(bb, [dbs[k], dis[k], col + rr], v)

    u0 = wid * UNITS_PW
    start_idx(u0, 0)
    wait_idx(0)
    for j in range(3):
        start_gather(0, j, j)

    def outer(k2, carry):
        for p in range(2):
            k = k2 * 2 + p
            u = u0 + k

            @pl.when(k + 1 < UNITS_PW)
            def _():
                start_idx(u + 1, 1 - p)

            @pl.when(k >= 2)
            def _():
                wait_out(p)

            for sub in range(NG):
                aslot = sub % NA
                pre = sub + 3
                if pre < NG:
                    start_gather(p, pre, pre % NA)
                else:
                    @pl.when(k + 1 < UNITS_PW)
                    def _(pre=pre):
                        if pre == NG:
                            wait_idx(1 - p)
                        start_gather(1 - p, pre - NG, pre % NA)

                wait_gather(aslot)
                transpose_block(aslot, p, sub)

            start_out(u, p)
        return carry

    lax.fori_loop(0, UNITS_PW // 2, outer, 0)

    for j in (UNITS_PW - 2, UNITS_PW - 1):
        wait_out(j % 2)


def kernel(x, table):
    xq = (
        x.astype(jnp.int32)
        .reshape(BB, 128, SB, 8)
        .transpose(2, 0, 3, 1)
    )
    t128 = _prep_table(table.T)
    out5 = _gather_t(xq, t128)
    return out5.transpose(2, 4, 0, 1, 3).reshape(BATCH, SEQ, DIM)


# speedup vs baseline: 2.4923x; 1.0378x over previous
import functools
import math

import jax
import jax.numpy as jnp
from jax import lax
from jax.experimental import pallas as pl
from jax.experimental.pallas import tpu as pltpu
from jax.experimental.pallas import tpu_sc as plsc

VOCAB = 1000000
DIM = 64
BATCH = 4096
SEQ = 200
SCALE = math.sqrt(DIM)

NC = 2
NS = 16
NW = NC * NS
LANES = 16
KSUB = DIM // LANES
PDIM = 128

TBLK = 16384
NTBLK = (VOCAB + TBLK - 1) // TBLK

BBLK = 512
NE = BATCH // BBLK
N_UNITS = SEQ * NE
UNITS_PW = N_UNITS // NW
GSUB = 64
NG = BBLK // GSUB
NA = 4
BPITCH = BBLK + 1
SB = SEQ // 8
BB = BATCH // 128


def _prep_table_block(tt_ref, out_ref):
    out_ref[:, 0:DIM] = jnp.transpose(tt_ref[...]) * SCALE


_prep_table = pl.pallas_call(
    _prep_table_block,
    grid=(NTBLK,),
    in_specs=[pl.BlockSpec((DIM, TBLK), lambda i: (0, i))],
    out_specs=pl.BlockSpec((TBLK, PDIM), lambda i: (i, 0)),
    out_shape=jax.ShapeDtypeStruct((VOCAB, PDIM), jnp.float32),
)


def _iota16():
    return lax.iota(jnp.int32, LANES)


@functools.partial(
    pl.kernel,
    out_type=jax.ShapeDtypeStruct((SEQ, 8, BB, 8, 128), jnp.float32),
    mesh=plsc.VectorSubcoreMesh(core_axis_name="c", subcore_axis_name="s"),
    scratch_types=[
        [pltpu.VMEM((BBLK,), jnp.int32) for _ in range(2)],
        [pltpu.VMEM((GSUB, PDIM), jnp.float32) for _ in range(NA)],
        [pltpu.VMEM((8, 8, BPITCH), jnp.float32) for _ in range(2)],
        [pltpu.SemaphoreType.DMA for _ in range(2)],
        [pltpu.SemaphoreType.DMA for _ in range(NA)],
        [pltpu.SemaphoreType.DMA for _ in range(2)],
    ],
    compiler_params=pltpu.CompilerParams(use_tc_tiling_on_sc=False,
                                         needs_layout_passes=False),
)
def _gather_t(xq_hbm, table_hbm, out_hbm, idxbufs, abufs, bbufs,
              isems, gsems, osems):
    wid = lax.axis_index("s") * NC + lax.axis_index("c")

    def unit_su(u):
        return u // NE, u % NE

    def start_idx(u, slot):
        s, e = unit_su(u)
        sb, si = s // 8, s % 8
        for q in range(4):
            pltpu.async_copy(
                xq_hbm.at[sb, e * 4 + q, si, :],
                idxbufs[slot].at[pl.ds(q * 128, 128)],
                isems[slot],
            )

    def wait_idx(slot):
        for _ in range(4):
            pltpu.make_async_copy(
                xq_hbm.at[0, 0, 0, :],
                idxbufs[slot].at[pl.ds(0, 128)],
                isems[slot],
            ).wait()

    def start_gather(islot, sub, aslot):
        pltpu.async_copy(
            table_hbm.at[idxbufs[islot].at[pl.ds(sub * GSUB, GSUB)]],
            abufs[aslot],
            gsems[aslot],
        )

    def wait_gather(aslot):
        pltpu.make_async_copy(
            table_hbm.at[idxbufs[0].at[pl.ds(0, GSUB)]],
            abufs[aslot],
            gsems[aslot],
        ).wait()

    def start_out(u, bslot):
        s, e = unit_su(u)
        for q in range(4):
            pltpu.async_copy(
                bbufs[bslot].at[:, :, pl.ds(q * 128, 128)],
                out_hbm.at[s, :, e * 4 + q, :, :],
                osems[bslot],
            )

    def wait_out(bslot):
        for _ in range(4):
            pltpu.make_async_copy(
                bbufs[bslot].at[:, :, pl.ds(0, 128)],
                out_hbm.at[0, :, 0, :, :],
                osems[bslot],
            ).wait()

    iota = _iota16()
    dbs = [iota // 8 + 2 * k for k in range(KSUB)]
    dis = [iota % 8 for _ in range(KSUB)]

    def transpose_block(aslot, bslot, sub):
        ab = abufs[aslot]
        bb = bbufs[bslot]
        base = jnp.full((LANES,), sub * GSUB, jnp.int32)

        @plsc.parallel_loop(0, GSUB, step=4, unroll=2)
        def _body(r):
            col = base + r
            for rr in range(4):
                for k in range(KSUB):
                    v = ab[r + rr, pl.ds(k * LANES, LANES)]
                    plsc.store_scatter(bb, [dbs[k], dis[k], col + rr], v)

    u0 = wid * UNITS_PW
    start_idx(u0, 0)
    wait_idx(0)
    for j in range(3):
        start_gather(0, j, j)

    def outer(k2, carry):
        for p in range(2):
            k = k2 * 2 + p
            u = u0 + k

            @pl.when(k + 1 < UNITS_PW)
            def _():
                start_idx(u + 1, 1 - p)

            @pl.when(k >= 2)
            def _():
                wait_out(p)

            for sub in range(NG):
                aslot = sub % NA
                pre = sub + 3
                if pre < NG:
                    start_gather(p, pre, pre % NA)
                else:
                    @pl.when(k + 1 < UNITS_PW)
                    def _(pre=pre):
                        if pre == NG:
                            wait_idx(1 - p)
                        start_gather(1 - p, pre - NG, pre % NA)

                wait_gather(aslot)
                transpose_block(aslot, p, sub)

            start_out(u, p)
        return carry

    lax.fori_loop(0, UNITS_PW // 2, outer, 0)

    for j in (UNITS_PW - 2, UNITS_PW - 1):
        wait_out(j % 2)


def kernel(x, table):
    xq = (
        x.astype(jnp.int32)
        .reshape(BB, 128, SB, 8)
        .transpose(2, 0, 3, 1)
    )
    t128 = _prep_table(table.T)
    out5 = _gather_t(xq, t128)
    return out5.transpose(2, 4, 0, 1, 3).reshape(BATCH, SEQ, DIM)
